# Initial kernel scaffold; baseline (speedup 1.0000x reference)
import jax
import jax.numpy as jnp
from jax.experimental import pallas as pl


def kernel(x, edge_index, pos, batch, t, params):
    raise NotImplementedError("write your pallas kernel here")


# speedup vs baseline: 1.7374x; 1.7374x over previous
import functools

import jax
import jax.numpy as jnp
import numpy as np
from jax import lax
from jax.experimental import pallas as pl
from jax.experimental.pallas import tpu as pltpu
from jax.experimental.pallas import tpu_sc as plsc

AD = 128
H = 256
HH = H // 2
TD = 128
N = 10000
E = 320000
NG = 256
NPAD = 10240
NC, NS = 2, 16
NW = NC * NS

BN = 400
BE = 512

_f32 = jnp.float32


def _silu(x):
    return x * jax.nn.sigmoid(x)


def _wspec(r, c):
    return pl.BlockSpec((r, c), lambda i: (0, 0))


def _tc_prep0(x, We, be, Wd, Ws):
    def body(x_ref, we_ref, be_ref, wd_ref, ws_ref, h_ref, td_ref, ts_ref):
        h = jnp.dot(x_ref[...], we_ref[...], precision=lax.Precision.HIGHEST, preferred_element_type=_f32) + be_ref[...]
        h_ref[...] = h
        td_ref[...] = jnp.dot(h, wd_ref[...], precision=lax.Precision.HIGHEST, preferred_element_type=_f32)
        ts_ref[...] = jnp.dot(h, ws_ref[...], precision=lax.Precision.HIGHEST, preferred_element_type=_f32)

    nb = N // BN
    return pl.pallas_call(
        body,
        grid=(nb,),
        in_specs=[
            pl.BlockSpec((BN, AD), lambda i: (i, 0)),
            _wspec(AD, H), _wspec(1, H), _wspec(H, H), _wspec(H, H),
        ],
        out_specs=[
            pl.BlockSpec((BN, H), lambda i: (i, 0)),
            pl.BlockSpec((BN, H), lambda i: (i, 0)),
            pl.BlockSpec((BN, H), lambda i: (i, 0)),
        ],
        out_shape=[
            jax.ShapeDtypeStruct((N, H), _f32),
            jax.ShapeDtypeStruct((N, H), _f32),
            jax.ShapeDtypeStruct((N, H), _f32),
        ],
    )(x, We, be, Wd, Ws)


def _tc_edge(gd, gs, d2c, w1l, b1, W2, b2, Wc1, bc1, wc2t, bc2, last):
    def body(gd_ref, gs_ref, d2_ref, w1l_ref, b1_ref, w2_ref, b2_ref,
             wc1_ref, bc1_ref, wc2_ref, bc2_ref, oa_ref, ob_ref, *rest):
        g = gd_ref[...] + gs_ref[...]
        d2 = d2_ref[...]
        t1 = _silu(g + d2 * w1l_ref[...] + b1_ref[...])
        m = _silu(jnp.dot(t1, w2_ref[...], precision=lax.Precision.HIGHEST, preferred_element_type=_f32) + b2_ref[...])
        oa_ref[...] = m[:, :HH]
        ob_ref[...] = m[:, HH:]
        if not last:
            c = _silu(jnp.dot(m, wc1_ref[...], precision=lax.Precision.HIGHEST, preferred_element_type=_f32) + bc1_ref[...])
            rest[0][...] = jnp.sum(c * wc2_ref[...], axis=1, keepdims=True) + bc2_ref[...]

    nb = E // BE
    out_specs = [
        pl.BlockSpec((BE, HH), lambda i: (i, 0)),
        pl.BlockSpec((BE, HH), lambda i: (i, 0)),
    ]
    out_shape = [
        jax.ShapeDtypeStruct((E, HH), _f32),
        jax.ShapeDtypeStruct((E, HH), _f32),
    ]
    if not last:
        out_specs.append(pl.BlockSpec((BE, 1), lambda i: (i, 0)))
        out_shape.append(jax.ShapeDtypeStruct((E, 1), _f32))
    return pl.pallas_call(
        body,
        grid=(nb,),
        in_specs=[
            pl.BlockSpec((BE, H), lambda i: (i, 0)),
            pl.BlockSpec((BE, H), lambda i: (i, 0)),
            pl.BlockSpec((BE, 1), lambda i: (i, 0)),
            _wspec(1, H), _wspec(1, H), _wspec(H, H), _wspec(1, H),
            _wspec(H, H), _wspec(1, H), _wspec(1, H), _wspec(1, 1),
        ],
        out_specs=out_specs,
        out_shape=out_shape,
    )(gd, gs, d2c, w1l, b1, W2, b2, Wc1, bc1, wc2t, bc2)


def _tc_node(h, pos4, sa, sb, pd4, n1a, n1b, bn1, n2, bn2, Wd, Ws, last):
    def body(*refs):
        if last:
            (h_ref, sa_ref, sb_ref, n1a_ref, n1b_ref, bn1_ref,
             n2_ref, bn2_ref, hn_ref) = refs
        else:
            (h_ref, p_ref, sa_ref, sb_ref, pd_ref, n1a_ref, n1b_ref, bn1_ref,
             n2_ref, bn2_ref, wd_ref, ws_ref,
             hn_ref, tdn_ref, tsn_ref, pn_ref) = refs
        hv = h_ref[...]
        agg = jnp.concatenate([sa_ref[...], sb_ref[...]], axis=1)
        u = _silu(jnp.dot(hv, n1a_ref[...], precision=lax.Precision.HIGHEST, preferred_element_type=_f32)
                  + jnp.dot(agg, n1b_ref[...], precision=lax.Precision.HIGHEST, preferred_element_type=_f32)
                  + bn1_ref[...])
        hn = hv + jnp.dot(u, n2_ref[...], precision=lax.Precision.HIGHEST, preferred_element_type=_f32) + bn2_ref[...]
        hn_ref[...] = hn
        if not last:
            pd = jnp.sum(pd_ref[...], axis=0)
            deg = jnp.maximum(pd[:, 3:4], 1.0)
            col = lax.broadcasted_iota(jnp.int32, (BN, 4), 1)
            pn_ref[...] = p_ref[...] + jnp.where(col < 3, pd, 0.0) / deg
            tdn_ref[...] = jnp.dot(hn, wd_ref[...], precision=lax.Precision.HIGHEST, preferred_element_type=_f32)
            tsn_ref[...] = jnp.dot(hn, ws_ref[...], precision=lax.Precision.HIGHEST, preferred_element_type=_f32)

    nb = N // BN
    if last:
        in_specs = [
            pl.BlockSpec((BN, H), lambda i: (i, 0)),
            pl.BlockSpec((BN, HH), lambda i: (i, 0)),
            pl.BlockSpec((BN, HH), lambda i: (i, 0)),
            _wspec(H, H), _wspec(H, H), _wspec(1, H), _wspec(H, H), _wspec(1, H),
        ]
        args = [h, sa, sb, n1a, n1b, bn1, n2, bn2]
        out_specs = [pl.BlockSpec((BN, H), lambda i: (i, 0))]
        out_shape = [jax.ShapeDtypeStruct((N, H), _f32)]
    else:
        in_specs = [
            pl.BlockSpec((BN, H), lambda i: (i, 0)),
            pl.BlockSpec((BN, 4), lambda i: (i, 0)),
            pl.BlockSpec((BN, HH), lambda i: (i, 0)),
            pl.BlockSpec((BN, HH), lambda i: (i, 0)),
            pl.BlockSpec((NW, BN, 4), lambda i: (0, i, 0)),
            _wspec(H, H), _wspec(H, H), _wspec(1, H), _wspec(H, H), _wspec(1, H),
            _wspec(H, H), _wspec(H, H),
        ]
        args = [h, pos4, sa, sb, pd4, n1a, n1b, bn1, n2, bn2, Wd, Ws]
        out_specs = [
            pl.BlockSpec((BN, H), lambda i: (i, 0)),
            pl.BlockSpec((BN, H), lambda i: (i, 0)),
            pl.BlockSpec((BN, H), lambda i: (i, 0)),
            pl.BlockSpec((BN, 4), lambda i: (i, 0)),
        ]
        out_shape = [
            jax.ShapeDtypeStruct((N, H), _f32),
            jax.ShapeDtypeStruct((N, H), _f32),
            jax.ShapeDtypeStruct((N, H), _f32),
            jax.ShapeDtypeStruct((N, 4), _f32),
        ]
    return pl.pallas_call(
        body, grid=(nb,), in_specs=in_specs, out_specs=out_specs,
        out_shape=out_shape)(*args)


def _tc_temb_table(tf, fr):
    def body(t_ref, f_ref, o_ref):
        args = t_ref[...] * f_ref[...]
        col = lax.broadcasted_iota(jnp.int32, (NG, TD), 1)
        o_ref[...] = jnp.where(col < TD // 2, jnp.sin(args), jnp.cos(args))

    return pl.pallas_call(
        body,
        grid=(1,),
        in_specs=[_wspec(NG, 1), _wspec(1, TD)],
        out_specs=pl.BlockSpec((NG, TD), lambda i: (0, 0)),
        out_shape=jax.ShapeDtypeStruct((NG, TD), _f32),
    )(tf, fr)


def _tc_final(h4, temb, Wg, bg, np1a, np1b, bnp1, np2, bnp2, np3p, bnp3):
    OW = 2 * AD

    def body(h_ref, te_ref, wg_ref, bg_ref, a_ref, b_ref, b1_ref,
             w2_ref, b2_ref, w3_ref, b3_ref, o_ref):
        nf = jnp.dot(h_ref[...], wg_ref[...], precision=lax.Precision.HIGHEST, preferred_element_type=_f32) + bg_ref[...]
        u = _silu(jnp.dot(nf, a_ref[...], precision=lax.Precision.HIGHEST, preferred_element_type=_f32)
                  + jnp.dot(te_ref[...], b_ref[...], precision=lax.Precision.HIGHEST, preferred_element_type=_f32)
                  + b1_ref[...])
        u = _silu(jnp.dot(u, w2_ref[...], precision=lax.Precision.HIGHEST, preferred_element_type=_f32) + b2_ref[...])
        o_ref[...] = jnp.dot(u, w3_ref[...], precision=lax.Precision.HIGHEST, preferred_element_type=_f32) + b3_ref[...]

    nb = N // BN
    return pl.pallas_call(
        body,
        grid=(nb,),
        in_specs=[
            pl.BlockSpec((BN, H), lambda i: (i, 0)),
            pl.BlockSpec((BN, TD), lambda i: (i, 0)),
            _wspec(H, H), _wspec(1, H), _wspec(H, H), _wspec(TD, H),
            _wspec(1, H), _wspec(H, H), _wspec(1, H), _wspec(H, OW), _wspec(1, OW),
        ],
        out_specs=pl.BlockSpec((BN, OW), lambda i: (i, 0)),
        out_shape=jax.ShapeDtypeStruct((N, OW), _f32),
    )(h4, temb, Wg, bg, np1a, np1b, bnp1, np2, bnp2, np3p, bnp3)


def _sc_mesh():
    return plsc.VectorSubcoreMesh(
        core_axis_name="c", subcore_axis_name="s",
        num_cores=NC, num_subcores=NS)


_SC_PARAMS = pltpu.CompilerParams(needs_layout_passes=False)


def _sc_gather_pair(td, ts, p4f, dst, src):
    C = 128
    EW = E // NW
    NCH = EW // C
    TAIL = EW - NCH * C
    N4 = N * 4

    @functools.partial(
        pl.kernel,
        out_type=(jax.ShapeDtypeStruct((E, H), _f32),
                  jax.ShapeDtypeStruct((E, H), _f32),
                  jax.ShapeDtypeStruct((E,), _f32)),
        mesh=_sc_mesh(),
        compiler_params=_SC_PARAMS,
        scratch_types=[
            pltpu.VMEM((C,), jnp.int32), pltpu.VMEM((C,), jnp.int32),
            pltpu.VMEM((C, H), _f32), pltpu.VMEM((C, H), _f32),
            pltpu.VMEM((C,), _f32),
            pltpu.VMEM((N4,), _f32),
            pltpu.SemaphoreType.DMA, pltpu.SemaphoreType.DMA,
        ],
    )
    def k(td_hbm, ts_hbm, p4_hbm, dst_hbm, src_hbm, od_hbm, os_hbm, d2_hbm,
          idxd, idxs, bufd, bufs, d2b, ptab, sem1, sem2):
        wid = lax.axis_index("s") * NC + lax.axis_index("c")
        base0 = wid * EW
        pltpu.sync_copy(p4_hbm, ptab)

        def chunk(base, n, iidxd, iidxs, ibufd, ibufs, id2b):
            pltpu.sync_copy(dst_hbm.at[pl.ds(base, n)], iidxd)
            pltpu.sync_copy(src_hbm.at[pl.ds(base, n)], iidxs)
            cp1 = pltpu.async_copy(td_hbm.at[iidxd], ibufd, sem1)
            cp2 = pltpu.async_copy(ts_hbm.at[iidxs], ibufs, sem2)
            for j in range(n // 16):
                dst16 = iidxd[pl.ds(j * 16, 16)] * 4
                src16 = iidxs[pl.ds(j * 16, 16)] * 4
                d2 = jnp.zeros((16,), _f32)
                for comp in range(3):
                    pdc = plsc.load_gather(ptab, [dst16 + comp])
                    psc = plsc.load_gather(ptab, [src16 + comp])
                    rel = pdc - psc
                    d2 = d2 + rel * rel
                id2b[pl.ds(j * 16, 16)] = d2
            cp1.wait()
            cp2.wait()
            pltpu.sync_copy(ibufd, od_hbm.at[pl.ds(base, n)])
            pltpu.sync_copy(ibufs, os_hbm.at[pl.ds(base, n)])
            pltpu.sync_copy(id2b, d2_hbm.at[pl.ds(base, n)])

        @pl.loop(0, NCH)
        def _(ch):
            chunk(base0 + ch * C, C, idxd, idxs, bufd, bufs, d2b)

        chunk(base0 + NCH * C, TAIL,
              idxd.at[pl.ds(0, TAIL)], idxs.at[pl.ds(0, TAIL)],
              bufd.at[pl.ds(0, TAIL)], bufs.at[pl.ds(0, TAIL)],
              d2b.at[pl.ds(0, TAIL)])

    return k(td, ts, p4f, dst, src)


def _sc_scatter_sum(oa, ob, dst, z2d):
    C = 128
    EW = E // NS
    NCH = EW // C
    TAIL = EW - NCH * C

    @functools.partial(
        pl.kernel,
        out_type=(jax.ShapeDtypeStruct((N, HH), _f32),
                  jax.ShapeDtypeStruct((N, HH), _f32)),
        mesh=_sc_mesh(),
        compiler_params=_SC_PARAMS,
        scratch_types=[
            pltpu.VMEM((C,), jnp.int32), pltpu.VMEM((C, HH), _f32),
            pltpu.VMEM((TAIL,), jnp.int32), pltpu.VMEM((TAIL, HH), _f32),
            pltpu.VMEM_SHARED((N, HH), _f32),
        ],
    )
    def k(oa_hbm, ob_hbm, dst_hbm, z2_hbm, sa_hbm, sb_hbm,
          idx, buf, idxt, buft, acc):
        c = lax.axis_index("c")
        s = lax.axis_index("s")
        Z0, Z1 = 624, N - 15 * 624

        @pl.when(s < 15)
        def _():
            pltpu.sync_copy(z2_hbm.at[pl.ds(0, Z0)], acc.at[pl.ds(s * Z0, Z0)])

        @pl.when(s == 15)
        def _():
            pltpu.sync_copy(z2_hbm, acc.at[pl.ds(15 * Z0, Z1)])

        plsc.subcore_barrier()

        def run(src_mat):
            def chunk(base, n, iidx, ibuf):
                pltpu.sync_copy(dst_hbm.at[pl.ds(base, n)], iidx)
                pltpu.sync_copy(src_mat.at[pl.ds(base, n)], ibuf)
                pltpu.sync_copy(ibuf, acc.at[iidx], add=True)

            @pl.loop(0, NCH)
            def _(ch):
                chunk(s * EW + ch * C, C, idx, buf)

            chunk(s * EW + NCH * C, TAIL, idxt, buft)

        @pl.when(c == 0)
        def _():
            run(oa_hbm)

        @pl.when(c == 1)
        def _():
            run(ob_hbm)

        plsc.subcore_barrier()

        def wb(out_hbm):
            @pl.when(s < 15)
            def _():
                pltpu.sync_copy(acc.at[pl.ds(s * Z0, Z0)],
                                out_hbm.at[pl.ds(s * Z0, Z0)])

            @pl.when(s == 15)
            def _():
                pltpu.sync_copy(acc.at[pl.ds(15 * Z0, Z1)],
                                out_hbm.at[pl.ds(15 * Z0, Z1)])

        @pl.when(c == 0)
        def _():
            wb(sa_hbm)

        @pl.when(c == 1)
        def _():
            wb(sb_hbm)

    return k(oa, ob, dst, z2d)


def _sc_pos_scatter(cwf, p4f, dst, src, zf):
    C = 128
    EW = E // NW
    NCH = EW // C
    TAIL = EW - NCH * C
    N4 = N * 4

    @functools.partial(
        pl.kernel,
        out_type=jax.ShapeDtypeStruct((NW * N4,), _f32),
        mesh=_sc_mesh(),
        compiler_params=_SC_PARAMS,
        scratch_types=[
            pltpu.VMEM((C,), jnp.int32), pltpu.VMEM((C,), jnp.int32),
            pltpu.VMEM((C,), _f32),
            pltpu.VMEM((N4,), _f32), pltpu.VMEM((N4,), _f32),
        ],
    )
    def k(cw_hbm, p4_hbm, dst_hbm, src_hbm, zf_hbm, pd_hbm,
          idxd, idxs, cwb, pacc, ptab):
        wid = lax.axis_index("s") * NC + lax.axis_index("c")
        base0 = wid * EW
        pltpu.sync_copy(zf_hbm, pacc)
        pltpu.sync_copy(p4_hbm, ptab)

        def chunk(base, n, iidxd, iidxs, icwb):
            pltpu.sync_copy(dst_hbm.at[pl.ds(base, n)], iidxd)
            pltpu.sync_copy(src_hbm.at[pl.ds(base, n)], iidxs)
            pltpu.sync_copy(cw_hbm.at[pl.ds(base, n)], icwb)
            for j in range(n // 16):
                dst16 = iidxd[pl.ds(j * 16, 16)] * 4
                src16 = iidxs[pl.ds(j * 16, 16)] * 4
                cw16 = icwb[pl.ds(j * 16, 16)]
                for comp in range(3):
                    pdc = plsc.load_gather(ptab, [dst16 + comp])
                    psc = plsc.load_gather(ptab, [src16 + comp])
                    plsc.addupdate_scatter(
                        pacc, [dst16 + comp], (pdc - psc) * cw16)
                plsc.addupdate_scatter(
                    pacc, [dst16 + 3], jnp.full((16,), 1.0, _f32))

        @pl.loop(0, NCH)
        def _(ch):
            chunk(base0 + ch * C, C, idxd, idxs, cwb)

        chunk(base0 + NCH * C, TAIL,
              idxd.at[pl.ds(0, TAIL)], idxs.at[pl.ds(0, TAIL)],
              cwb.at[pl.ds(0, TAIL)])

        pltpu.sync_copy(pacc, pd_hbm.at[pl.ds(wid * N4, N4)])

    return k(cwf, p4f, dst, src, zf)


def _sc_temb_gather(table, batchp):
    C = 128
    RW = NPAD // NW
    NCH = RW // C
    TAIL = RW - NCH * C

    @functools.partial(
        pl.kernel,
        out_type=jax.ShapeDtypeStruct((NPAD, TD), _f32),
        mesh=_sc_mesh(),
        compiler_params=_SC_PARAMS,
        scratch_types=[
            pltpu.VMEM((C,), jnp.int32), pltpu.VMEM((C, TD), _f32),
            pltpu.SemaphoreType.DMA,
        ],
    )
    def k(tab_hbm, idx_hbm, out_hbm, idx, buf, sem):
        wid = lax.axis_index("s") * NC + lax.axis_index("c")
        base0 = wid * RW

        @pl.loop(0, NCH)
        def _(ch):
            base = base0 + ch * C
            pltpu.sync_copy(idx_hbm.at[pl.ds(base, C)], idx)
            pltpu.async_copy(tab_hbm.at[idx], buf, sem).wait()
            pltpu.sync_copy(buf, out_hbm.at[pl.ds(base, C)])

        base = base0 + NCH * C
        it = idx.at[pl.ds(0, TAIL)]
        bt = buf.at[pl.ds(0, TAIL)]
        pltpu.sync_copy(idx_hbm.at[pl.ds(base, TAIL)], it)
        pltpu.async_copy(tab_hbm.at[it], bt, sem).wait()
        pltpu.sync_copy(bt, out_hbm.at[pl.ds(base, TAIL)])

    return k(table, batchp)


def kernel(x, edge_index, pos, batch, t, params):
    src = edge_index[0].astype(jnp.int32)
    dst = edge_index[1].astype(jnp.int32)
    p4f = jnp.pad(pos.astype(_f32), ((0, 0), (0, 1))).reshape(-1)
    batchp = jnp.pad(batch.astype(jnp.int32), (0, NPAD - N))
    zf = jnp.zeros((N * 4,), _f32)
    z2d = jnp.zeros((640, HH), _f32)

    def w(name):
        return params[name]["w"]

    def b2d(name):
        return params[name]["b"].reshape(1, -1)

    half = TD // 2
    freqs = np.exp(-np.log(10000.0) * np.arange(half, dtype=np.float32) / (half - 1))
    fr = jnp.asarray(np.concatenate([freqs, freqs])[None, :], _f32)
    table = _tc_temb_table(t.astype(_f32).reshape(NG, 1), fr)
    tembn = _sc_temb_gather(table, batchp)[:N]

    W1 = [w(f"edge{l}_1") for l in range(4)]
    h, td, ts = _tc_prep0(x, w("embed"), b2d("embed"), W1[0][:H], W1[0][H:2 * H])

    h4 = None
    for l in range(4):
        last = l == 3
        gd, gs, d2f = _sc_gather_pair(td, ts, p4f, dst, src)
        eo = _tc_edge(
            gd, gs, d2f.reshape(E, 1),
            W1[l][2 * H:2 * H + 1], b2d(f"edge{l}_1"),
            w(f"edge{l}_2"), b2d(f"edge{l}_2"),
            w(f"coord{l}_1"), b2d(f"coord{l}_1"),
            w(f"coord{l}_2").reshape(1, H), params[f"coord{l}_2"]["b"].reshape(1, 1),
            last)
        n1 = w(f"node{l}_1")
        if last:
            oa, ob = eo
            sa, sb = _sc_scatter_sum(oa, ob, dst, z2d)
            h4 = _tc_node(h, None, sa, sb, None, n1[:H], n1[H:],
                          b2d(f"node{l}_1"), w(f"node{l}_2"), b2d(f"node{l}_2"),
                          None, None, True)[0]
        else:
            oa, ob, cw = eo
            sa, sb = _sc_scatter_sum(oa, ob, dst, z2d)
            pdf = _sc_pos_scatter(cw.reshape(-1), p4f, dst, src, zf)
            h, td, ts, pos4 = _tc_node(
                h, p4f.reshape(N, 4), sa, sb, pdf.reshape(NW, N, 4),
                n1[:H], n1[H:], b2d(f"node{l}_1"),
                w(f"node{l}_2"), b2d(f"node{l}_2"),
                W1[l + 1][:H], W1[l + 1][H:2 * H], False)
            p4f = pos4.reshape(-1)

    np1 = w("np1")
    np3p = jnp.pad(w("np3"), ((0, 0), (0, 2 * AD - AD - 3)))
    bnp3 = jnp.pad(b2d("np3"), ((0, 0), (0, 2 * AD - AD - 3)))
    out = _tc_final(h4, tembn, w("gnn_out"), b2d("gnn_out"),
                    np1[:H], np1[H:], b2d("np1"),
                    w("np2"), b2d("np2"), np3p, bnp3)
    return out[:, :AD], out[:, AD:AD + 3]


# speedup vs baseline: 2.1601x; 1.2433x over previous
import functools

import jax
import jax.numpy as jnp
import numpy as np
from jax import lax
from jax.experimental import pallas as pl
from jax.experimental.pallas import tpu as pltpu
from jax.experimental.pallas import tpu_sc as plsc

AD = 128
H = 256
HH = H // 2
TD = 128
N = 10000
E = 320000
NG = 256
NPAD = 10240
NC, NS = 2, 16
NW = NC * NS

BN = 400
BE = 512

_f32 = jnp.float32


def _silu(x):
    return x * jax.nn.sigmoid(x)


def _wspec(r, c):
    return pl.BlockSpec((r, c), lambda i: (0, 0))


def _tc_prep0(x, We, be, Wd, Ws):
    def body(x_ref, we_ref, be_ref, wd_ref, ws_ref, h_ref, td_ref, ts_ref):
        h = jnp.dot(x_ref[...], we_ref[...], preferred_element_type=_f32) + be_ref[...]
        h_ref[...] = h
        td_ref[...] = jnp.dot(h, wd_ref[...], preferred_element_type=_f32)
        ts_ref[...] = jnp.dot(h, ws_ref[...], preferred_element_type=_f32)

    nb = N // BN
    return pl.pallas_call(
        body,
        grid=(nb,),
        in_specs=[
            pl.BlockSpec((BN, AD), lambda i: (i, 0)),
            _wspec(AD, H), _wspec(1, H), _wspec(H, H), _wspec(H, H),
        ],
        out_specs=[
            pl.BlockSpec((BN, H), lambda i: (i, 0)),
            pl.BlockSpec((BN, H), lambda i: (i, 0)),
            pl.BlockSpec((BN, H), lambda i: (i, 0)),
        ],
        out_shape=[
            jax.ShapeDtypeStruct((N, H), _f32),
            jax.ShapeDtypeStruct((N, H), _f32),
            jax.ShapeDtypeStruct((N, H), _f32),
        ],
    )(x, We, be, Wd, Ws)


def _tc_edge(gd, gs, rel8, w1l, b1, W2, b2, Wc1, bc1, wc2t, bc2, last):
    def body(gd_ref, gs_ref, r_ref, w1l_ref, b1_ref, w2_ref, b2_ref,
             wc1_ref, bc1_ref, wc2_ref, bc2_ref, oa_ref, ob_ref, *rest):
        g = gd_ref[...] + gs_ref[...]
        rel3 = r_ref[:, :3]
        d2 = jnp.sum(rel3 * rel3, axis=1, keepdims=True)
        d2b = d2.astype(jnp.bfloat16).astype(_f32)
        w1lb = w1l_ref[...].astype(jnp.bfloat16).astype(_f32)
        t1 = _silu(g + d2b * w1lb + b1_ref[...])
        m = _silu(jnp.dot(t1, w2_ref[...], preferred_element_type=_f32) + b2_ref[...])
        oa_ref[...] = m[:, :HH]
        ob_ref[...] = m[:, HH:]
        if not last:
            c = _silu(jnp.dot(m, wc1_ref[...], preferred_element_type=_f32) + bc1_ref[...])
            cb = c.astype(jnp.bfloat16).astype(_f32)
            wcb = wc2_ref[...].astype(jnp.bfloat16).astype(_f32)
            rest[0][...] = jnp.sum(cb * wcb, axis=1, keepdims=True) + bc2_ref[...]

    nb = E // BE
    out_specs = [
        pl.BlockSpec((BE, HH), lambda i: (i, 0)),
        pl.BlockSpec((BE, HH), lambda i: (i, 0)),
    ]
    out_shape = [
        jax.ShapeDtypeStruct((E, HH), _f32),
        jax.ShapeDtypeStruct((E, HH), _f32),
    ]
    if not last:
        out_specs.append(pl.BlockSpec((BE, 1), lambda i: (i, 0)))
        out_shape.append(jax.ShapeDtypeStruct((E, 1), _f32))
    return pl.pallas_call(
        body,
        grid=(nb,),
        in_specs=[
            pl.BlockSpec((BE, H), lambda i: (i, 0)),
            pl.BlockSpec((BE, H), lambda i: (i, 0)),
            pl.BlockSpec((BE, 8), lambda i: (i, 0)),
            _wspec(1, H), _wspec(1, H), _wspec(H, H), _wspec(1, H),
            _wspec(H, H), _wspec(1, H), _wspec(1, H), _wspec(1, 1),
        ],
        out_specs=out_specs,
        out_shape=out_shape,
    )(gd, gs, rel8, w1l, b1, W2, b2, Wc1, bc1, wc2t, bc2)


def _tc_node(h, pos4, sa, sb, pd4, n1a, n1b, bn1, n2, bn2, Wd, Ws, last):
    def body(*refs):
        if last:
            (h_ref, sa_ref, sb_ref, n1a_ref, n1b_ref, bn1_ref,
             n2_ref, bn2_ref, hn_ref) = refs
        else:
            (h_ref, p_ref, sa_ref, sb_ref, pd_ref, n1a_ref, n1b_ref, bn1_ref,
             n2_ref, bn2_ref, wd_ref, ws_ref,
             hn_ref, tdn_ref, tsn_ref, pn_ref) = refs
        hv = h_ref[...]
        agg = jnp.concatenate([sa_ref[...], sb_ref[...]], axis=1)
        u = _silu(jnp.dot(hv, n1a_ref[...], preferred_element_type=_f32)
                  + jnp.dot(agg, n1b_ref[...], preferred_element_type=_f32)
                  + bn1_ref[...])
        hn = hv + jnp.dot(u, n2_ref[...], preferred_element_type=_f32) + bn2_ref[...]
        hn_ref[...] = hn
        if not last:
            pd = jnp.sum(pd_ref[...], axis=0)
            deg = jnp.maximum(pd[:, 3:4], 1.0)
            col = lax.broadcasted_iota(jnp.int32, (BN, 4), 1)
            pn_ref[...] = p_ref[...] + jnp.where(col < 3, pd, 0.0) / deg
            tdn_ref[...] = jnp.dot(hn, wd_ref[...], preferred_element_type=_f32)
            tsn_ref[...] = jnp.dot(hn, ws_ref[...], preferred_element_type=_f32)

    nb = N // BN
    if last:
        in_specs = [
            pl.BlockSpec((BN, H), lambda i: (i, 0)),
            pl.BlockSpec((BN, HH), lambda i: (i, 0)),
            pl.BlockSpec((BN, HH), lambda i: (i, 0)),
            _wspec(H, H), _wspec(H, H), _wspec(1, H), _wspec(H, H), _wspec(1, H),
        ]
        args = [h, sa, sb, n1a, n1b, bn1, n2, bn2]
        out_specs = [pl.BlockSpec((BN, H), lambda i: (i, 0))]
        out_shape = [jax.ShapeDtypeStruct((N, H), _f32)]
    else:
        in_specs = [
            pl.BlockSpec((BN, H), lambda i: (i, 0)),
            pl.BlockSpec((BN, 4), lambda i: (i, 0)),
            pl.BlockSpec((BN, HH), lambda i: (i, 0)),
            pl.BlockSpec((BN, HH), lambda i: (i, 0)),
            pl.BlockSpec((NW, BN, 4), lambda i: (0, i, 0)),
            _wspec(H, H), _wspec(H, H), _wspec(1, H), _wspec(H, H), _wspec(1, H),
            _wspec(H, H), _wspec(H, H),
        ]
        args = [h, pos4, sa, sb, pd4, n1a, n1b, bn1, n2, bn2, Wd, Ws]
        out_specs = [
            pl.BlockSpec((BN, H), lambda i: (i, 0)),
            pl.BlockSpec((BN, H), lambda i: (i, 0)),
            pl.BlockSpec((BN, H), lambda i: (i, 0)),
            pl.BlockSpec((BN, 4), lambda i: (i, 0)),
        ]
        out_shape = [
            jax.ShapeDtypeStruct((N, H), _f32),
            jax.ShapeDtypeStruct((N, H), _f32),
            jax.ShapeDtypeStruct((N, H), _f32),
            jax.ShapeDtypeStruct((N, 4), _f32),
        ]
    return pl.pallas_call(
        body, grid=(nb,), in_specs=in_specs, out_specs=out_specs,
        out_shape=out_shape)(*args)


def _tc_temb_table(tf, fr):
    def body(t_ref, f_ref, o_ref):
        args = t_ref[...] * f_ref[...]
        col = lax.broadcasted_iota(jnp.int32, (NG, TD), 1)
        o_ref[...] = jnp.where(col < TD // 2, jnp.sin(args), jnp.cos(args))

    return pl.pallas_call(
        body,
        grid=(1,),
        in_specs=[_wspec(NG, 1), _wspec(1, TD)],
        out_specs=pl.BlockSpec((NG, TD), lambda i: (0, 0)),
        out_shape=jax.ShapeDtypeStruct((NG, TD), _f32),
    )(tf, fr)


def _tc_final(h4, temb, Wg, bg, np1a, np1b, bnp1, np2, bnp2, np3p, bnp3):
    OW = 2 * AD

    def body(h_ref, te_ref, wg_ref, bg_ref, a_ref, b_ref, b1_ref,
             w2_ref, b2_ref, w3_ref, b3_ref, o_ref):
        nf = jnp.dot(h_ref[...], wg_ref[...], preferred_element_type=_f32) + bg_ref[...]
        u = _silu(jnp.dot(nf, a_ref[...], preferred_element_type=_f32)
                  + jnp.dot(te_ref[...], b_ref[...], preferred_element_type=_f32)
                  + b1_ref[...])
        u = _silu(jnp.dot(u, w2_ref[...], preferred_element_type=_f32) + b2_ref[...])
        o_ref[...] = jnp.dot(u, w3_ref[...], preferred_element_type=_f32) + b3_ref[...]

    nb = N // BN
    return pl.pallas_call(
        body,
        grid=(nb,),
        in_specs=[
            pl.BlockSpec((BN, H), lambda i: (i, 0)),
            pl.BlockSpec((BN, TD), lambda i: (i, 0)),
            _wspec(H, H), _wspec(1, H), _wspec(H, H), _wspec(TD, H),
            _wspec(1, H), _wspec(H, H), _wspec(1, H), _wspec(H, OW), _wspec(1, OW),
        ],
        out_specs=pl.BlockSpec((BN, OW), lambda i: (i, 0)),
        out_shape=jax.ShapeDtypeStruct((N, OW), _f32),
    )(h4, temb, Wg, bg, np1a, np1b, bnp1, np2, bnp2, np3p, bnp3)


def _sc_mesh():
    return plsc.VectorSubcoreMesh(
        core_axis_name="c", subcore_axis_name="s",
        num_cores=NC, num_subcores=NS)


_SC_PARAMS = pltpu.CompilerParams(needs_layout_passes=False)


def _sc_gather_pair(td, ts, p4f, dst, src):
    C = 128
    EW = E // NW
    NCH = EW // C
    TAIL = EW - NCH * C
    N4 = N * 4

    @functools.partial(
        pl.kernel,
        out_type=(jax.ShapeDtypeStruct((E, H), _f32),
                  jax.ShapeDtypeStruct((E, H), _f32),
                  jax.ShapeDtypeStruct((E * 8,), _f32)),
        mesh=_sc_mesh(),
        compiler_params=_SC_PARAMS,
        scratch_types=[
            pltpu.VMEM((C,), jnp.int32), pltpu.VMEM((C,), jnp.int32),
            pltpu.VMEM((C, H), _f32), pltpu.VMEM((C, H), _f32),
            pltpu.VMEM((C * 8,), _f32),
            pltpu.VMEM((N4,), _f32),
            pltpu.SemaphoreType.DMA, pltpu.SemaphoreType.DMA,
        ],
    )
    def k(td_hbm, ts_hbm, p4_hbm, dst_hbm, src_hbm, od_hbm, os_hbm, r8_hbm,
          idxd, idxs, bufd, bufs, r8b, ptab, sem1, sem2):
        wid = lax.axis_index("s") * NC + lax.axis_index("c")
        base0 = wid * EW
        pltpu.sync_copy(p4_hbm, ptab)
        lane = lax.iota(jnp.int32, 16)

        def chunk(base, n, iidxd, iidxs, ibufd, ibufs, ir8b):
            pltpu.sync_copy(dst_hbm.at[pl.ds(base, n)], iidxd)
            pltpu.sync_copy(src_hbm.at[pl.ds(base, n)], iidxs)
            cp1 = pltpu.async_copy(td_hbm.at[iidxd], ibufd, sem1)
            cp2 = pltpu.async_copy(ts_hbm.at[iidxs], ibufs, sem2)
            for j in range(n // 16):
                dst16 = iidxd[pl.ds(j * 16, 16)] * 4
                src16 = iidxs[pl.ds(j * 16, 16)] * 4
                flat = (lane + j * 16) * 8
                for comp in range(3):
                    pdc = plsc.load_gather(ptab, [dst16 + comp])
                    psc = plsc.load_gather(ptab, [src16 + comp])
                    plsc.store_scatter(ir8b, [flat + comp], pdc - psc)
            cp1.wait()
            cp2.wait()
            pltpu.sync_copy(ibufd, od_hbm.at[pl.ds(base, n)])
            pltpu.sync_copy(ibufs, os_hbm.at[pl.ds(base, n)])
            pltpu.sync_copy(ir8b, r8_hbm.at[pl.ds(base * 8, n * 8)])

        @pl.loop(0, NCH)
        def _(ch):
            chunk(base0 + ch * C, C, idxd, idxs, bufd, bufs, r8b)

        chunk(base0 + NCH * C, TAIL,
              idxd.at[pl.ds(0, TAIL)], idxs.at[pl.ds(0, TAIL)],
              bufd.at[pl.ds(0, TAIL)], bufs.at[pl.ds(0, TAIL)],
              r8b.at[pl.ds(0, TAIL * 8)])

    return k(td, ts, p4f, dst, src)


def _sc_scatter_sum(oa, ob, dst, z2d):
    C = 128
    EW = E // NS
    NCH = EW // C
    TAIL = EW - NCH * C

    @functools.partial(
        pl.kernel,
        out_type=(jax.ShapeDtypeStruct((N, HH), _f32),
                  jax.ShapeDtypeStruct((N, HH), _f32)),
        mesh=_sc_mesh(),
        compiler_params=_SC_PARAMS,
        scratch_types=[
            pltpu.VMEM((C,), jnp.int32), pltpu.VMEM((C, HH), _f32),
            pltpu.VMEM((TAIL,), jnp.int32), pltpu.VMEM((TAIL, HH), _f32),
            pltpu.VMEM_SHARED((N, HH), _f32),
        ],
    )
    def k(oa_hbm, ob_hbm, dst_hbm, z2_hbm, sa_hbm, sb_hbm,
          idx, buf, idxt, buft, acc):
        c = lax.axis_index("c")
        s = lax.axis_index("s")
        Z0, Z1 = 624, N - 15 * 624

        @pl.when(s < 15)
        def _():
            pltpu.sync_copy(z2_hbm.at[pl.ds(0, Z0)], acc.at[pl.ds(s * Z0, Z0)])

        @pl.when(s == 15)
        def _():
            pltpu.sync_copy(z2_hbm, acc.at[pl.ds(15 * Z0, Z1)])

        plsc.subcore_barrier()

        def run(src_mat):
            def chunk(base, n, iidx, ibuf):
                pltpu.sync_copy(dst_hbm.at[pl.ds(base, n)], iidx)
                pltpu.sync_copy(src_mat.at[pl.ds(base, n)], ibuf)
                pltpu.sync_copy(ibuf, acc.at[iidx], add=True)

            @pl.loop(0, NCH)
            def _(ch):
                chunk(s * EW + ch * C, C, idx, buf)

            chunk(s * EW + NCH * C, TAIL, idxt, buft)

        @pl.when(c == 0)
        def _():
            run(oa_hbm)

        @pl.when(c == 1)
        def _():
            run(ob_hbm)

        plsc.subcore_barrier()

        def wb(out_hbm):
            @pl.when(s < 15)
            def _():
                pltpu.sync_copy(acc.at[pl.ds(s * Z0, Z0)],
                                out_hbm.at[pl.ds(s * Z0, Z0)])

            @pl.when(s == 15)
            def _():
                pltpu.sync_copy(acc.at[pl.ds(15 * Z0, Z1)],
                                out_hbm.at[pl.ds(15 * Z0, Z1)])

        @pl.when(c == 0)
        def _():
            wb(sa_hbm)

        @pl.when(c == 1)
        def _():
            wb(sb_hbm)

    return k(oa, ob, dst, z2d)


def _sc_pos_scatter(cwf, r8f, dst, zf):
    C = 128
    EW = E // NW
    NCH = EW // C
    TAIL = EW - NCH * C
    N4 = N * 4

    @functools.partial(
        pl.kernel,
        out_type=jax.ShapeDtypeStruct((NW * N4,), _f32),
        mesh=_sc_mesh(),
        compiler_params=_SC_PARAMS,
        scratch_types=[
            pltpu.VMEM((C,), jnp.int32), pltpu.VMEM((C,), _f32),
            pltpu.VMEM((C * 8,), _f32),
            pltpu.VMEM((N4,), _f32),
        ],
    )
    def k(cw_hbm, r8_hbm, dst_hbm, zf_hbm, pd_hbm, idxd, cwb, r8b, pacc):
        wid = lax.axis_index("s") * NC + lax.axis_index("c")
        base0 = wid * EW
        pltpu.sync_copy(zf_hbm, pacc)
        lane = lax.iota(jnp.int32, 16)

        def chunk(base, n, iidxd, icwb, ir8b):
            pltpu.sync_copy(dst_hbm.at[pl.ds(base, n)], iidxd)
            pltpu.sync_copy(cw_hbm.at[pl.ds(base, n)], icwb)
            pltpu.sync_copy(r8_hbm.at[pl.ds(base * 8, n * 8)], ir8b)
            for j in range(n // 16):
                dst16 = iidxd[pl.ds(j * 16, 16)] * 4
                cw16 = icwb[pl.ds(j * 16, 16)]
                flat = (lane + j * 16) * 8
                for comp in range(3):
                    rel = plsc.load_gather(ir8b, [flat + comp])
                    plsc.addupdate_scatter(pacc, [dst16 + comp], rel * cw16)
                plsc.addupdate_scatter(
                    pacc, [dst16 + 3], jnp.full((16,), 1.0, _f32))

        @pl.loop(0, NCH)
        def _(ch):
            chunk(base0 + ch * C, C, idxd, cwb, r8b)

        chunk(base0 + NCH * C, TAIL,
              idxd.at[pl.ds(0, TAIL)], cwb.at[pl.ds(0, TAIL)],
              r8b.at[pl.ds(0, TAIL * 8)])

        pltpu.sync_copy(pacc, pd_hbm.at[pl.ds(wid * N4, N4)])

    return k(cwf, r8f, dst, zf)


def _sc_temb_gather(table, batchp):
    C = 128
    RW = NPAD // NW
    NCH = RW // C
    TAIL = RW - NCH * C

    @functools.partial(
        pl.kernel,
        out_type=jax.ShapeDtypeStruct((NPAD, TD), _f32),
        mesh=_sc_mesh(),
        compiler_params=_SC_PARAMS,
        scratch_types=[
            pltpu.VMEM((C,), jnp.int32), pltpu.VMEM((C, TD), _f32),
            pltpu.SemaphoreType.DMA,
        ],
    )
    def k(tab_hbm, idx_hbm, out_hbm, idx, buf, sem):
        wid = lax.axis_index("s") * NC + lax.axis_index("c")
        base0 = wid * RW

        @pl.loop(0, NCH)
        def _(ch):
            base = base0 + ch * C
            pltpu.sync_copy(idx_hbm.at[pl.ds(base, C)], idx)
            pltpu.async_copy(tab_hbm.at[idx], buf, sem).wait()
            pltpu.sync_copy(buf, out_hbm.at[pl.ds(base, C)])

        base = base0 + NCH * C
        it = idx.at[pl.ds(0, TAIL)]
        bt = buf.at[pl.ds(0, TAIL)]
        pltpu.sync_copy(idx_hbm.at[pl.ds(base, TAIL)], it)
        pltpu.async_copy(tab_hbm.at[it], bt, sem).wait()
        pltpu.sync_copy(bt, out_hbm.at[pl.ds(base, TAIL)])

    return k(table, batchp)


def kernel(x, edge_index, pos, batch, t, params):
    src = edge_index[0].astype(jnp.int32)
    dst = edge_index[1].astype(jnp.int32)
    p4f = jnp.pad(pos.astype(_f32), ((0, 0), (0, 1))).reshape(-1)
    batchp = jnp.pad(batch.astype(jnp.int32), (0, NPAD - N))
    zf = jnp.zeros((N * 4,), _f32)
    z2d = jnp.zeros((640, HH), _f32)

    def w(name):
        return params[name]["w"]

    def b2d(name):
        return params[name]["b"].reshape(1, -1)

    half = TD // 2
    freqs = np.exp(-np.log(10000.0) * np.arange(half, dtype=np.float32) / (half - 1))
    fr = jnp.asarray(np.concatenate([freqs, freqs])[None, :], _f32)
    table = _tc_temb_table(t.astype(_f32).reshape(NG, 1), fr)
    tembn = _sc_temb_gather(table, batchp)[:N]

    W1 = [w(f"edge{l}_1") for l in range(4)]
    h, td, ts = _tc_prep0(x, w("embed"), b2d("embed"), W1[0][:H], W1[0][H:2 * H])

    h4 = None
    for l in range(4):
        last = l == 3
        gd, gs, r8f = _sc_gather_pair(td, ts, p4f, dst, src)
        eo = _tc_edge(
            gd, gs, r8f.reshape(E, 8),
            W1[l][2 * H:2 * H + 1], b2d(f"edge{l}_1"),
            w(f"edge{l}_2"), b2d(f"edge{l}_2"),
            w(f"coord{l}_1"), b2d(f"coord{l}_1"),
            w(f"coord{l}_2").reshape(1, H), params[f"coord{l}_2"]["b"].reshape(1, 1),
            last)
        n1 = w(f"node{l}_1")
        if last:
            oa, ob = eo
            sa, sb = _sc_scatter_sum(oa, ob, dst, z2d)
            h4 = _tc_node(h, None, sa, sb, None, n1[:H], n1[H:],
                          b2d(f"node{l}_1"), w(f"node{l}_2"), b2d(f"node{l}_2"),
                          None, None, True)[0]
        else:
            oa, ob, cw = eo
            sa, sb = _sc_scatter_sum(oa, ob, dst, z2d)
            pdf = _sc_pos_scatter(cw.reshape(-1), r8f, dst, zf)
            h, td, ts, pos4 = _tc_node(
                h, p4f.reshape(N, 4), sa, sb, pdf.reshape(NW, N, 4),
                n1[:H], n1[H:], b2d(f"node{l}_1"),
                w(f"node{l}_2"), b2d(f"node{l}_2"),
                W1[l + 1][:H], W1[l + 1][H:2 * H], False)
            p4f = pos4.reshape(-1)

    np1 = w("np1")
    np3p = jnp.pad(w("np3"), ((0, 0), (0, 2 * AD - AD - 3)))
    bnp3 = jnp.pad(b2d("np3"), ((0, 0), (0, 2 * AD - AD - 3)))
    out = _tc_final(h4, tembn, w("gnn_out"), b2d("gnn_out"),
                    np1[:H], np1[H:], b2d("np1"),
                    w("np2"), b2d("np2"), np3p, bnp3)
    return out[:, :AD], out[:, AD:AD + 3]


# speedup vs baseline: 2.2015x; 1.0192x over previous
import functools

import jax
import jax.numpy as jnp
import numpy as np
from jax import lax
from jax.experimental import pallas as pl
from jax.experimental.pallas import tpu as pltpu
from jax.experimental.pallas import tpu_sc as plsc

AD = 128
H = 256
HH = H // 2
TD = 128
N = 10000
E = 320000
NG = 256
NPAD = 10240
NC, NS = 2, 16
NW = NC * NS

BN = 400
BE = 512

_f32 = jnp.float32


def _silu(x):
    return x * jax.nn.sigmoid(x)


def _wspec(r, c):
    return pl.BlockSpec((r, c), lambda i: (0, 0))


def _tc_embed(x, We, be):
    def body(x_ref, we_ref, be_ref, h_ref):
        h_ref[...] = jnp.dot(x_ref[...], we_ref[...], preferred_element_type=_f32) + be_ref[...]

    nb = N // BN
    return pl.pallas_call(
        body,
        grid=(nb,),
        in_specs=[
            pl.BlockSpec((BN, AD), lambda i: (i, 0)),
            _wspec(AD, H), _wspec(1, H),
        ],
        out_specs=pl.BlockSpec((BN, H), lambda i: (i, 0)),
        out_shape=jax.ShapeDtypeStruct((N, H), _f32),
    )(x, We, be)


def _tc_edge(gd, gs, rel8, W1, b1, W2, b2, Wc1, bc1, wc2, bc2, last):
    def body(gd_ref, gs_ref, r_ref, w1_ref, b1_ref, w2_ref, b2_ref,
             wc1_ref, bc1_ref, wc2_ref, bc2_ref, oa_ref, ob_ref, *rest):
        rel3 = r_ref[:, :3]
        d2 = jnp.sum(rel3 * rel3, axis=1, keepdims=True)
        e_in = jnp.concatenate([gd_ref[...], gs_ref[...], d2], axis=1)
        t1 = _silu(jnp.dot(e_in, w1_ref[...], preferred_element_type=_f32) + b1_ref[...])
        m = _silu(jnp.dot(t1, w2_ref[...], preferred_element_type=_f32) + b2_ref[...])
        oa_ref[...] = m[:, :HH]
        ob_ref[...] = m[:, HH:]
        if not last:
            c = _silu(jnp.dot(m, wc1_ref[...], preferred_element_type=_f32) + bc1_ref[...])
            rest[0][...] = jnp.dot(c, wc2_ref[...], preferred_element_type=_f32) + bc2_ref[...]

    nb = E // BE
    out_specs = [
        pl.BlockSpec((BE, HH), lambda i: (i, 0)),
        pl.BlockSpec((BE, HH), lambda i: (i, 0)),
    ]
    out_shape = [
        jax.ShapeDtypeStruct((E, HH), _f32),
        jax.ShapeDtypeStruct((E, HH), _f32),
    ]
    if not last:
        out_specs.append(pl.BlockSpec((BE, 1), lambda i: (i, 0)))
        out_shape.append(jax.ShapeDtypeStruct((E, 1), _f32))
    return pl.pallas_call(
        body,
        grid=(nb,),
        in_specs=[
            pl.BlockSpec((BE, H), lambda i: (i, 0)),
            pl.BlockSpec((BE, H), lambda i: (i, 0)),
            pl.BlockSpec((BE, 8), lambda i: (i, 0)),
            _wspec(2 * H + 1, H), _wspec(1, H), _wspec(H, H), _wspec(1, H),
            _wspec(H, H), _wspec(1, H), _wspec(H, 1), _wspec(1, 1),
        ],
        out_specs=out_specs,
        out_shape=out_shape,
    )(gd, gs, rel8, W1, b1, W2, b2, Wc1, bc1, wc2, bc2)


def _tc_node(h, pos4, sa, sb, pd4, n1, bn1, n2, bn2, last):
    def body(*refs):
        if last:
            (h_ref, sa_ref, sb_ref, n1_ref, bn1_ref,
             n2_ref, bn2_ref, hn_ref) = refs
        else:
            (h_ref, p_ref, sa_ref, sb_ref, pd_ref, n1_ref, bn1_ref,
             n2_ref, bn2_ref, hn_ref, pn_ref) = refs
        hv = h_ref[...]
        n_in = jnp.concatenate([hv, sa_ref[...], sb_ref[...]], axis=1)
        u = _silu(jnp.dot(n_in, n1_ref[...], preferred_element_type=_f32)
                  + bn1_ref[...])
        hn_ref[...] = hv + jnp.dot(u, n2_ref[...], preferred_element_type=_f32) + bn2_ref[...]
        if not last:
            pd = jnp.sum(pd_ref[...], axis=0)
            deg = jnp.maximum(pd[:, 3:4], 1.0)
            col = lax.broadcasted_iota(jnp.int32, (BN, 4), 1)
            pn_ref[...] = p_ref[...] + jnp.where(col < 3, pd, 0.0) / deg

    nb = N // BN
    if last:
        in_specs = [
            pl.BlockSpec((BN, H), lambda i: (i, 0)),
            pl.BlockSpec((BN, HH), lambda i: (i, 0)),
            pl.BlockSpec((BN, HH), lambda i: (i, 0)),
            _wspec(2 * H, H), _wspec(1, H), _wspec(H, H), _wspec(1, H),
        ]
        args = [h, sa, sb, n1, bn1, n2, bn2]
        out_specs = [pl.BlockSpec((BN, H), lambda i: (i, 0))]
        out_shape = [jax.ShapeDtypeStruct((N, H), _f32)]
    else:
        in_specs = [
            pl.BlockSpec((BN, H), lambda i: (i, 0)),
            pl.BlockSpec((BN, 4), lambda i: (i, 0)),
            pl.BlockSpec((BN, HH), lambda i: (i, 0)),
            pl.BlockSpec((BN, HH), lambda i: (i, 0)),
            pl.BlockSpec((NW, BN, 4), lambda i: (0, i, 0)),
            _wspec(2 * H, H), _wspec(1, H), _wspec(H, H), _wspec(1, H),
        ]
        args = [h, pos4, sa, sb, pd4, n1, bn1, n2, bn2]
        out_specs = [
            pl.BlockSpec((BN, H), lambda i: (i, 0)),
            pl.BlockSpec((BN, 4), lambda i: (i, 0)),
        ]
        out_shape = [
            jax.ShapeDtypeStruct((N, H), _f32),
            jax.ShapeDtypeStruct((N, 4), _f32),
        ]
    return pl.pallas_call(
        body, grid=(nb,), in_specs=in_specs, out_specs=out_specs,
        out_shape=out_shape)(*args)


def _tc_temb_table(tf, fr):
    def body(t_ref, f_ref, o_ref):
        args = t_ref[...] * f_ref[...]
        col = lax.broadcasted_iota(jnp.int32, (NG, TD), 1)
        o_ref[...] = jnp.where(col < TD // 2, jnp.sin(args), jnp.cos(args))

    return pl.pallas_call(
        body,
        grid=(1,),
        in_specs=[_wspec(NG, 1), _wspec(1, TD)],
        out_specs=pl.BlockSpec((NG, TD), lambda i: (0, 0)),
        out_shape=jax.ShapeDtypeStruct((NG, TD), _f32),
    )(tf, fr)


def _tc_final(h4, temb, Wg, bg, np1, bnp1, np2, bnp2, np3p, bnp3):
    OW = 2 * AD

    def body(h_ref, te_ref, wg_ref, bg_ref, w1_ref, b1_ref,
             w2_ref, b2_ref, w3_ref, b3_ref, o_ref):
        nf = jnp.dot(h_ref[...], wg_ref[...], preferred_element_type=_f32) + bg_ref[...]
        comb = jnp.concatenate([nf, te_ref[...]], axis=1)
        u = _silu(jnp.dot(comb, w1_ref[...], preferred_element_type=_f32) + b1_ref[...])
        u = _silu(jnp.dot(u, w2_ref[...], preferred_element_type=_f32) + b2_ref[...])
        o_ref[...] = jnp.dot(u, w3_ref[...], preferred_element_type=_f32) + b3_ref[...]

    nb = N // BN
    return pl.pallas_call(
        body,
        grid=(nb,),
        in_specs=[
            pl.BlockSpec((BN, H), lambda i: (i, 0)),
            pl.BlockSpec((BN, TD), lambda i: (i, 0)),
            _wspec(H, H), _wspec(1, H), _wspec(H + TD, H),
            _wspec(1, H), _wspec(H, H), _wspec(1, H), _wspec(H, OW), _wspec(1, OW),
        ],
        out_specs=pl.BlockSpec((BN, OW), lambda i: (i, 0)),
        out_shape=jax.ShapeDtypeStruct((N, OW), _f32),
    )(h4, temb, Wg, bg, np1, bnp1, np2, bnp2, np3p, bnp3)


def _sc_mesh():
    return plsc.VectorSubcoreMesh(
        core_axis_name="c", subcore_axis_name="s",
        num_cores=NC, num_subcores=NS)


_SC_PARAMS = pltpu.CompilerParams(needs_layout_passes=False)


def _sc_gather_pair(td, ts, p4f, dst, src):
    C = 64
    EW = E // NW
    NCH = EW // C
    NIT = NCH // 2
    TAIL = EW - NCH * C
    N4 = N * 4

    @functools.partial(
        pl.kernel,
        out_type=(jax.ShapeDtypeStruct((E, H), _f32),
                  jax.ShapeDtypeStruct((E, H), _f32),
                  jax.ShapeDtypeStruct((E * 8,), _f32)),
        mesh=_sc_mesh(),
        compiler_params=_SC_PARAMS,
        scratch_types=[
            pltpu.VMEM((C,), jnp.int32), pltpu.VMEM((C,), jnp.int32),
            pltpu.VMEM((C,), jnp.int32), pltpu.VMEM((C,), jnp.int32),
            pltpu.VMEM((C, H), _f32), pltpu.VMEM((C, H), _f32),
            pltpu.VMEM((C, H), _f32), pltpu.VMEM((C, H), _f32),
            pltpu.VMEM((C * 8,), _f32), pltpu.VMEM((C * 8,), _f32),
            pltpu.VMEM((N4,), _f32),
            pltpu.SemaphoreType.DMA, pltpu.SemaphoreType.DMA,
            pltpu.SemaphoreType.DMA, pltpu.SemaphoreType.DMA,
            pltpu.SemaphoreType.DMA, pltpu.SemaphoreType.DMA,
        ],
    )
    def k(td_hbm, ts_hbm, p4_hbm, dst_hbm, src_hbm, od_hbm, os_hbm, r8_hbm,
          idxd0, idxs0, idxd1, idxs1, bufd0, bufs0, bufd1, bufs1,
          r8b0, r8b1, ptab, semi0, semi1, semg0, semg1, semw0, semw1):
        wid = lax.axis_index("s") * NC + lax.axis_index("c")
        base0 = wid * EW
        pltpu.sync_copy(p4_hbm, ptab)
        lane = lax.iota(jnp.int32, 16)

        sets = ((idxd0, idxs0, bufd0, bufs0, r8b0, semi0, semg0, semw0),
                (idxd1, idxs1, bufd1, bufs1, r8b1, semi1, semg1, semw1))

        def issue_idx(b, base):
            idxd, idxs, _, _, _, semi, _, _ = sets[b]
            pltpu.async_copy(dst_hbm.at[pl.ds(base, C)], idxd, semi)
            pltpu.async_copy(src_hbm.at[pl.ds(base, C)], idxs, semi)

        def drain_idx(b):
            idxd, idxs, _, _, _, semi, _, _ = sets[b]
            pltpu.make_async_copy(dst_hbm.at[pl.ds(0, C)], idxd, semi).wait()
            pltpu.make_async_copy(src_hbm.at[pl.ds(0, C)], idxs, semi).wait()

        def issue_gather(b):
            idxd, idxs, bufd, bufs, _, _, semg, _ = sets[b]
            pltpu.async_copy(td_hbm.at[idxd], bufd, semg)
            pltpu.async_copy(ts_hbm.at[idxs], bufs, semg)

        def drain_gather(b):
            idxd, idxs, bufd, bufs, _, _, semg, _ = sets[b]
            pltpu.make_async_copy(td_hbm.at[idxd], bufd, semg).wait()
            pltpu.make_async_copy(ts_hbm.at[idxs], bufs, semg).wait()

        def drain_write(b):
            _, _, bufd, bufs, r8b, _, _, semw = sets[b]
            pltpu.make_async_copy(bufd, od_hbm.at[pl.ds(0, C)], semw).wait()
            pltpu.make_async_copy(bufs, os_hbm.at[pl.ds(0, C)], semw).wait()
            pltpu.make_async_copy(r8b, r8_hbm.at[pl.ds(0, C * 8)], semw).wait()

        def compute_store(b, base):
            idxd, idxs, bufd, bufs, r8b, _, _, semw = sets[b]
            for j in range(C // 16):
                dst16 = idxd[pl.ds(j * 16, 16)] * 4
                src16 = idxs[pl.ds(j * 16, 16)] * 4
                flat = (lane + j * 16) * 8
                for comp in range(3):
                    pdc = plsc.load_gather(ptab, [dst16 + comp])
                    psc = plsc.load_gather(ptab, [src16 + comp])
                    plsc.store_scatter(r8b, [flat + comp], pdc - psc)

            pltpu.async_copy(bufd, od_hbm.at[pl.ds(base, C)], semw)
            pltpu.async_copy(bufs, os_hbm.at[pl.ds(base, C)], semw)
            pltpu.async_copy(r8b, r8_hbm.at[pl.ds(base * 8, C * 8)], semw)

        issue_idx(0, base0)
        issue_idx(1, base0 + C)
        drain_idx(0)
        issue_gather(0)
        drain_idx(1)
        issue_gather(1)

        @pl.loop(0, NIT - 1)
        def _(it):
            cur = base0 + 2 * it * C
            nxt = cur + 2 * C
            drain_gather(0)
            compute_store(0, cur)
            issue_idx(0, nxt)
            drain_gather(1)
            compute_store(1, cur + C)
            issue_idx(1, nxt + C)
            drain_idx(0)
            drain_write(0)
            issue_gather(0)
            drain_idx(1)
            drain_write(1)
            issue_gather(1)

        last = base0 + (NCH - 2) * C
        drain_gather(0)
        compute_store(0, last)
        drain_gather(1)
        compute_store(1, last + C)
        drain_write(0)
        drain_write(1)

        base = base0 + NCH * C
        it0 = idxd0.at[pl.ds(0, TAIL)]
        is0 = idxs0.at[pl.ds(0, TAIL)]
        bd0 = bufd0.at[pl.ds(0, TAIL)]
        bs0 = bufs0.at[pl.ds(0, TAIL)]
        r80 = r8b0.at[pl.ds(0, TAIL * 8)]
        pltpu.sync_copy(dst_hbm.at[pl.ds(base, TAIL)], it0)
        pltpu.sync_copy(src_hbm.at[pl.ds(base, TAIL)], is0)
        cp1 = pltpu.async_copy(td_hbm.at[it0], bd0, semg0)
        cp2 = pltpu.async_copy(ts_hbm.at[is0], bs0, semg1)
        cp1.wait()
        cp2.wait()
        dst16 = idxd0[pl.ds(0, 16)] * 4
        src16 = idxs0[pl.ds(0, 16)] * 4
        for comp in range(3):
            pdc = plsc.load_gather(ptab, [dst16 + comp])
            psc = plsc.load_gather(ptab, [src16 + comp])
            plsc.store_scatter(r8b0, [lane * 8 + comp], pdc - psc)

        pltpu.sync_copy(bd0, od_hbm.at[pl.ds(base, TAIL)])
        pltpu.sync_copy(bs0, os_hbm.at[pl.ds(base, TAIL)])
        pltpu.sync_copy(r80, r8_hbm.at[pl.ds(base * 8, TAIL * 8)])

    return k(td, ts, p4f, dst, src)


def _sc_scatter_sum(oa, ob, dst, z2d):
    C = 128
    EW = E // NS
    NCH = EW // C
    TAIL = EW - NCH * C

    @functools.partial(
        pl.kernel,
        out_type=(jax.ShapeDtypeStruct((N, HH), _f32),
                  jax.ShapeDtypeStruct((N, HH), _f32)),
        mesh=_sc_mesh(),
        compiler_params=_SC_PARAMS,
        scratch_types=[
            pltpu.VMEM((C,), jnp.int32), pltpu.VMEM((C, HH), _f32),
            pltpu.VMEM((TAIL,), jnp.int32), pltpu.VMEM((TAIL, HH), _f32),
            pltpu.VMEM_SHARED((N, HH), _f32),
        ],
    )
    def k(oa_hbm, ob_hbm, dst_hbm, z2_hbm, sa_hbm, sb_hbm,
          idx, buf, idxt, buft, acc):
        c = lax.axis_index("c")
        s = lax.axis_index("s")
        Z0, Z1 = 624, N - 15 * 624

        @pl.when(s < 15)
        def _():
            pltpu.sync_copy(z2_hbm.at[pl.ds(0, Z0)], acc.at[pl.ds(s * Z0, Z0)])

        @pl.when(s == 15)
        def _():
            pltpu.sync_copy(z2_hbm, acc.at[pl.ds(15 * Z0, Z1)])

        plsc.subcore_barrier()

        def run(src_mat):
            def chunk(base, n, iidx, ibuf):
                pltpu.sync_copy(dst_hbm.at[pl.ds(base, n)], iidx)
                pltpu.sync_copy(src_mat.at[pl.ds(base, n)], ibuf)
                pltpu.sync_copy(ibuf, acc.at[iidx], add=True)

            @pl.loop(0, NCH)
            def _(ch):
                chunk(s * EW + ch * C, C, idx, buf)

            chunk(s * EW + NCH * C, TAIL, idxt, buft)

        @pl.when(c == 0)
        def _():
            run(oa_hbm)

        @pl.when(c == 1)
        def _():
            run(ob_hbm)

        plsc.subcore_barrier()

        def wb(out_hbm):
            @pl.when(s < 15)
            def _():
                pltpu.sync_copy(acc.at[pl.ds(s * Z0, Z0)],
                                out_hbm.at[pl.ds(s * Z0, Z0)])

            @pl.when(s == 15)
            def _():
                pltpu.sync_copy(acc.at[pl.ds(15 * Z0, Z1)],
                                out_hbm.at[pl.ds(15 * Z0, Z1)])

        @pl.when(c == 0)
        def _():
            wb(sa_hbm)

        @pl.when(c == 1)
        def _():
            wb(sb_hbm)

    return k(oa, ob, dst, z2d)


def _sc_pos_scatter(cwf, r8f, dst, zf):
    C = 128
    EW = E // NW
    NCH = EW // C
    TAIL = EW - NCH * C
    N4 = N * 4

    @functools.partial(
        pl.kernel,
        out_type=jax.ShapeDtypeStruct((NW * N4,), _f32),
        mesh=_sc_mesh(),
        compiler_params=_SC_PARAMS,
        scratch_types=[
            pltpu.VMEM((C,), jnp.int32), pltpu.VMEM((C,), _f32),
            pltpu.VMEM((C * 8,), _f32),
            pltpu.VMEM((N4,), _f32),
        ],
    )
    def k(cw_hbm, r8_hbm, dst_hbm, zf_hbm, pd_hbm, idxd, cwb, r8b, pacc):
        wid = lax.axis_index("s") * NC + lax.axis_index("c")
        base0 = wid * EW
        pltpu.sync_copy(zf_hbm, pacc)
        lane = lax.iota(jnp.int32, 16)

        def chunk(base, n, iidxd, icwb, ir8b):
            pltpu.sync_copy(dst_hbm.at[pl.ds(base, n)], iidxd)
            pltpu.sync_copy(cw_hbm.at[pl.ds(base, n)], icwb)
            pltpu.sync_copy(r8_hbm.at[pl.ds(base * 8, n * 8)], ir8b)
            for j in range(n // 16):
                dst16 = iidxd[pl.ds(j * 16, 16)] * 4
                cw16 = icwb[pl.ds(j * 16, 16)]
                flat = (lane + j * 16) * 8
                for comp in range(3):
                    rel = plsc.load_gather(ir8b, [flat + comp])
                    plsc.addupdate_scatter(pacc, [dst16 + comp], rel * cw16)
                plsc.addupdate_scatter(
                    pacc, [dst16 + 3], jnp.full((16,), 1.0, _f32))

        @pl.loop(0, NCH)
        def _(ch):
            chunk(base0 + ch * C, C, idxd, cwb, r8b)

        chunk(base0 + NCH * C, TAIL,
              idxd.at[pl.ds(0, TAIL)], cwb.at[pl.ds(0, TAIL)],
              r8b.at[pl.ds(0, TAIL * 8)])

        pltpu.sync_copy(pacc, pd_hbm.at[pl.ds(wid * N4, N4)])

    return k(cwf, r8f, dst, zf)


def _sc_temb_gather(table, batchp):
    C = 128
    RW = NPAD // NW
    NCH = RW // C
    TAIL = RW - NCH * C

    @functools.partial(
        pl.kernel,
        out_type=jax.ShapeDtypeStruct((NPAD, TD), _f32),
        mesh=_sc_mesh(),
        compiler_params=_SC_PARAMS,
        scratch_types=[
            pltpu.VMEM((C,), jnp.int32), pltpu.VMEM((C, TD), _f32),
            pltpu.SemaphoreType.DMA,
        ],
    )
    def k(tab_hbm, idx_hbm, out_hbm, idx, buf, sem):
        wid = lax.axis_index("s") * NC + lax.axis_index("c")
        base0 = wid * RW

        @pl.loop(0, NCH)
        def _(ch):
            base = base0 + ch * C
            pltpu.sync_copy(idx_hbm.at[pl.ds(base, C)], idx)
            pltpu.async_copy(tab_hbm.at[idx], buf, sem).wait()
            pltpu.sync_copy(buf, out_hbm.at[pl.ds(base, C)])

        base = base0 + NCH * C
        it = idx.at[pl.ds(0, TAIL)]
        bt = buf.at[pl.ds(0, TAIL)]
        pltpu.sync_copy(idx_hbm.at[pl.ds(base, TAIL)], it)
        pltpu.async_copy(tab_hbm.at[it], bt, sem).wait()
        pltpu.sync_copy(bt, out_hbm.at[pl.ds(base, TAIL)])

    return k(table, batchp)


def kernel(x, edge_index, pos, batch, t, params):
    src = edge_index[0].astype(jnp.int32)
    dst = edge_index[1].astype(jnp.int32)
    p4f = jnp.pad(pos.astype(_f32), ((0, 0), (0, 1))).reshape(-1)
    batchp = jnp.pad(batch.astype(jnp.int32), (0, NPAD - N))
    zf = jnp.zeros((N * 4,), _f32)
    z2d = jnp.zeros((640, HH), _f32)

    def w(name):
        return params[name]["w"]

    def b2d(name):
        return params[name]["b"].reshape(1, -1)

    half = TD // 2
    freqs = np.exp(-np.log(10000.0) * np.arange(half, dtype=np.float32) / (half - 1))
    fr = jnp.asarray(np.concatenate([freqs, freqs])[None, :], _f32)
    table = _tc_temb_table(t.astype(_f32).reshape(NG, 1), fr)
    tembn = _sc_temb_gather(table, batchp)[:N]

    h = _tc_embed(x, w("embed"), b2d("embed"))

    h4 = None
    for l in range(4):
        last = l == 3
        gd, gs, r8f = _sc_gather_pair(h, h, p4f, dst, src)
        eo = _tc_edge(
            gd, gs, r8f.reshape(E, 8),
            w(f"edge{l}_1"), b2d(f"edge{l}_1"),
            w(f"edge{l}_2"), b2d(f"edge{l}_2"),
            w(f"coord{l}_1"), b2d(f"coord{l}_1"),
            w(f"coord{l}_2"), params[f"coord{l}_2"]["b"].reshape(1, 1),
            last)
        n1 = w(f"node{l}_1")
        if last:
            oa, ob = eo
            sa, sb = _sc_scatter_sum(oa, ob, dst, z2d)
            h4 = _tc_node(h, None, sa, sb, None, n1, b2d(f"node{l}_1"),
                          w(f"node{l}_2"), b2d(f"node{l}_2"), True)[0]
        else:
            oa, ob, cw = eo
            sa, sb = _sc_scatter_sum(oa, ob, dst, z2d)
            pdf = _sc_pos_scatter(cw.reshape(-1), r8f, dst, zf)
            h, pos4 = _tc_node(
                h, p4f.reshape(N, 4), sa, sb, pdf.reshape(NW, N, 4),
                n1, b2d(f"node{l}_1"),
                w(f"node{l}_2"), b2d(f"node{l}_2"), False)
            p4f = pos4.reshape(-1)

    np3p = jnp.pad(w("np3"), ((0, 0), (0, 2 * AD - AD - 3)))
    bnp3 = jnp.pad(b2d("np3"), ((0, 0), (0, 2 * AD - AD - 3)))
    out = _tc_final(h4, tembn, w("gnn_out"), b2d("gnn_out"),
                    w("np1"), b2d("np1"),
                    w("np2"), b2d("np2"), np3p, bnp3)
    return out[:, :AD], out[:, AD:AD + 3]
